# baseline (speedup 1.0000x reference)
import functools

import jax
import jax.numpy as jnp
from jax import lax
from jax.experimental import pallas as pl
from jax.experimental.pallas import tpu as pltpu
from jax.experimental.pallas import tpu_sc as plsc

N_DRUGS = 2000
N_NODES = 10000
H = 64
R = 53
E = 640000
H2 = 128

NN_PAD = 10240
NKEY_PAD = 530432
ZCH = NKEY_PAD // 16
NROW = 640
NC, NS = 2, 16
NW = NC * NS
EW = E // NW
WIN = 80
NWIN = EW // WIN
NPAIR = 27

_F32 = jnp.float32


def _sc_mesh():
    return plsc.VectorSubcoreMesh(
        core_axis_name="c", subcore_axis_name="s", num_cores=NC, num_subcores=NS
    )


def _mm_bias_body(x_ref, w_ref, b_ref, o_ref):
    o_ref[...] = (
        jnp.dot(x_ref[...], w_ref[...], preferred_element_type=_F32)
        + b_ref[...]
    )


def _mm_bias(x, w, b, bm):
    m, k = x.shape
    n = w.shape[1]
    return pl.pallas_call(
        _mm_bias_body,
        grid=(m // bm,),
        in_specs=[
            pl.BlockSpec((bm, k), lambda i: (i, 0)),
            pl.BlockSpec((k, n), lambda i: (0, 0)),
            pl.BlockSpec((1, n), lambda i: (0, 0)),
        ],
        out_specs=pl.BlockSpec((bm, n), lambda i: (i, 0)),
        out_shape=jax.ShapeDtypeStruct((m, n), _F32),
    )(x, w, b.reshape(1, n))


def _wmix_body(c_ref, b_ref, o_ref):
    o_ref[...] = jnp.dot(c_ref[...], b_ref[...], preferred_element_type=_F32)


def _wmix(compe, basise_flat):
    return pl.pallas_call(
        _wmix_body,
        out_shape=jax.ShapeDtypeStruct((NPAIR, H * H2), _F32),
    )(compe, basise_flat)


def _tables_body(w_ref, x_ref, o_ref):
    o_ref[0] = jnp.dot(x_ref[...], w_ref[0], preferred_element_type=_F32)


def _tables(x, w_all):
    return pl.pallas_call(
        _tables_body,
        grid=(NPAIR,),
        in_specs=[
            pl.BlockSpec((1, H, H2), lambda r: (r, 0, 0)),
            pl.BlockSpec((NN_PAD, H), lambda r: (0, 0)),
        ],
        out_specs=pl.BlockSpec((1, NN_PAD, H2), lambda r: (r, 0, 0)),
        out_shape=jax.ShapeDtypeStruct((NPAIR, NN_PAD, H2), _F32),
    )(w_all, x)


def _combine_body(acc_ref, yt_ref, b_ref, o_ref, *, relu):
    v = acc_ref[0] + acc_ref[1] + yt_ref[0, :, H:] + b_ref[...]
    o_ref[...] = jnp.maximum(v, 0.0) if relu else v


def _combine(acc, ytab, bias, relu):
    return pl.pallas_call(
        functools.partial(_combine_body, relu=relu),
        grid=(1,),
        in_specs=[
            pl.BlockSpec((2, NN_PAD, H), lambda i: (0, 0, 0)),
            pl.BlockSpec((1, NN_PAD, H2), lambda i: (NPAIR - 1, 0, 0)),
            pl.BlockSpec((1, H), lambda i: (0, 0)),
        ],
        out_specs=pl.BlockSpec((NN_PAD, H), lambda i: (0, 0)),
        out_shape=jax.ShapeDtypeStruct((NN_PAD, H), _F32),
    )(acc, ytab, bias.reshape(1, H))


def _sc_counts(dst, et):
    @functools.partial(
        pl.kernel,
        out_type=jax.ShapeDtypeStruct((NC, NKEY_PAD), _F32),
        mesh=_sc_mesh(),
        compiler_params=pltpu.CompilerParams(use_tc_tiling_on_sc=False),
        scratch_types=[
            pltpu.VMEM_SHARED((NKEY_PAD,), _F32),
            pltpu.VMEM((ZCH,), _F32),
        ]
        + [pltpu.VMEM((WIN,), jnp.int32) for _ in range(6)]
        + [pltpu.VMEM((WIN,), _F32)]
        + [pltpu.SemaphoreType.DMA for _ in range(4)],
    )
    def k(dst_hbm, et_hbm, out_hbm, cnt_sh, zb,
          dstb0, dstb1, etb0, etb1, keyb0, keyb1, onesb,
          sl0, sl1, ss0, ss1):
        cid = lax.axis_index("c")
        sid = lax.axis_index("s")
        wid = sid * NC + cid
        base = wid * EW

        dstb = (dstb0, dstb1)
        etb = (etb0, etb1)
        keyb = (keyb0, keyb1)
        sl = (sl0, sl1)
        ss = (ss0, ss1)

        z16 = jnp.zeros((16,), _F32)
        o16 = jnp.ones((16,), _F32)

        def zbody(i, _):
            zb[pl.ds(i * 16, 16)] = z16
            return 0

        lax.fori_loop(0, ZCH // 16, zbody, 0)
        pltpu.sync_copy(zb, cnt_sh.at[pl.ds(sid * ZCH, ZCH)])
        for t in range(WIN // 16):
            onesb[pl.ds(16 * t, 16)] = o16
        plsc.subcore_barrier()

        def fire_linear(w, p):
            o = base + jnp.minimum(w, NWIN - 1) * WIN
            pltpu.async_copy(dst_hbm.at[pl.ds(o, WIN)], dstb[p], sl[p])
            pltpu.async_copy(et_hbm.at[pl.ds(o, WIN)], etb[p], sl[p])

        def wait_linear(p):
            pltpu.make_async_copy(dst_hbm.at[pl.ds(base, WIN)], dstb[p], sl[p]).wait()
            pltpu.make_async_copy(et_hbm.at[pl.ds(base, WIN)], etb[p], sl[p]).wait()

        def wait_scatter(p):
            pltpu.make_async_copy(onesb, cnt_sh.at[keyb[p]], ss[p]).wait()

        fire_linear(0, 0)
        fire_linear(1, 1)

        def body2(w2, _):
            for p in (0, 1):
                w = w2 * 2 + p
                wait_linear(p)
                @pl.when(w >= 2)
                def _():
                    wait_scatter(p)
                for t in range(WIN // 16):
                    sli = pl.ds(16 * t, 16)
                    keyb[p][sli] = dstb[p][sli] * R + etb[p][sli]
                pltpu.async_copy(onesb, cnt_sh.at[keyb[p]], ss[p], add=True)
                fire_linear(w + 2, p)
            return 0

        lax.fori_loop(0, NWIN // 2, body2, 0)
        wait_linear(0)
        wait_linear(1)
        wait_scatter(0)
        wait_scatter(1)
        plsc.subcore_barrier()
        pltpu.sync_copy(
            cnt_sh.at[pl.ds(sid * ZCH, ZCH)],
            out_hbm.at[cid, pl.ds(sid * ZCH, ZCH)],
        )

    return k(dst, et)


def _sc_edges(ytab_flat, cnt, src, dst, et):
    @functools.partial(
        pl.kernel,
        out_type=jax.ShapeDtypeStruct((NC, NN_PAD, H), _F32),
        mesh=_sc_mesh(),
        compiler_params=pltpu.CompilerParams(use_tc_tiling_on_sc=False),
        scratch_types=[
            pltpu.VMEM_SHARED((NN_PAD, H), _F32),
            pltpu.VMEM((NROW, H), _F32),
        ]
        + [pltpu.VMEM((WIN,), jnp.int32) for _ in range(13)]
        + [pltpu.VMEM((WIN,), _F32) for _ in range(3)]
        + [pltpu.VMEM((WIN, H2), _F32) for _ in range(2)]
        + [pltpu.VMEM((WIN, H), _F32) for _ in range(2)]
        + [pltpu.SemaphoreType.DMA for _ in range(6)],
    )
    def k(yt_hbm, cnt_hbm, src_hbm, dst_hbm, et_hbm, out_hbm,
          acc_sh, zb,
          srcb0, srcb1, dstb0, dstb1, etb0, etb1,
          rowb0, rowb1, keyb0, keyb1, dsts0, dsts1, colb,
          cb0, cb1, invb, yb0, yb1, ysc0, ysc1,
          sl0, sl1, sg0, sg1, ss0, ss1):
        cid = lax.axis_index("c")
        sid = lax.axis_index("s")
        wid = sid * NC + cid
        base = wid * EW

        srcb = (srcb0, srcb1)
        dstb = (dstb0, dstb1)
        etb = (etb0, etb1)
        rowb = (rowb0, rowb1)
        keyb = (keyb0, keyb1)
        dsts = (dsts0, dsts1)
        cb = (cb0, cb1)
        yb = (yb0, yb1)
        ysc = (ysc0, ysc1)
        sl = (sl0, sl1)
        sg = (sg0, sg1)
        ss = (ss0, ss1)

        z16 = jnp.zeros((16,), _F32)
        zch = H // 16

        def zbody(i, _):
            zb[pl.ds(i // zch, 1), pl.ds((i % zch) * 16, 16)] = z16.reshape(1, 16)
            return 0

        lax.fori_loop(0, NROW * zch, zbody, 0)
        pltpu.sync_copy(zb, acc_sh.at[pl.ds(sid * NROW, NROW)])
        plsc.subcore_barrier()

        def off_(w):
            return base + jnp.minimum(w, NWIN - 1) * WIN

        def fire_linear(w, p):
            o = off_(w)
            pltpu.async_copy(src_hbm.at[pl.ds(o, WIN)], srcb[p], sl[p])
            pltpu.async_copy(dst_hbm.at[pl.ds(o, WIN)], dstb[p], sl[p])
            pltpu.async_copy(et_hbm.at[pl.ds(o, WIN)], etb[p], sl[p])

        def wait_linear(p):
            pltpu.make_async_copy(src_hbm.at[pl.ds(base, WIN)], srcb[p], sl[p]).wait()
            pltpu.make_async_copy(dst_hbm.at[pl.ds(base, WIN)], dstb[p], sl[p]).wait()
            pltpu.make_async_copy(et_hbm.at[pl.ds(base, WIN)], etb[p], sl[p]).wait()

        def keys_and_fire_gather(p):
            for t in range(WIN // 16):
                sli = pl.ds(16 * t, 16)
                e16 = etb[p][sli]
                rowb[p][sli] = (e16 >> 1) * NN_PAD + srcb[p][sli]
                keyb[p][sli] = dstb[p][sli] * R + e16
            pltpu.async_copy(cnt_hbm.at[keyb[p]], cb[p], sg[p])
            pltpu.async_copy(yt_hbm.at[rowb[p]], yb[p], sg[p])

        def wait_gather(p):
            pltpu.make_async_copy(cnt_hbm.at[keyb[p]], cb[p], sg[p]).wait()
            pltpu.make_async_copy(yt_hbm.at[rowb[p]], yb[p], sg[p]).wait()

        def wait_scatter(p):
            pltpu.make_async_copy(ysc[p], acc_sh.at[dsts[p]], ss[p]).wait()

        fire_linear(0, 0)
        wait_linear(0)
        keys_and_fire_gather(0)
        fire_linear(1, 1)

        def body2(w2, _):
            for p in (0, 1):
                w = w2 * 2 + p
                q = 1 - p
                wait_linear(q)
                keys_and_fire_gather(q)
                @pl.when(w >= 2)
                def _():
                    wait_scatter(p)
                wait_gather(p)
                for t in range(WIN // 16):
                    sli = pl.ds(16 * t, 16)
                    invb[sli] = 1.0 / jnp.maximum(cb[p][sli], 1.0)
                    dsts[p][sli] = dstb[p][sli]
                    colb[sli] = (etb[p][sli] & 1) * H

                def gbody(t, _):
                    iv16 = invb[pl.ds(16 * t, 16)]
                    co16 = colb[pl.ds(16 * t, 16)]
                    for l in range(16):
                        sc = iv16[l]
                        hi = co16[l] > 0
                        for j in range(H // 16):
                            sld = (pl.ds(16 * t + l, 1), pl.ds(16 * j, 16))
                            slh = (pl.ds(16 * t + l, 1), pl.ds(H + 16 * j, 16))
                            v = jnp.where(hi, yb[p][slh], yb[p][sld])
                            ysc[p][sld] = v * sc
                    return 0

                lax.fori_loop(0, WIN // 16, gbody, 0)
                pltpu.async_copy(ysc[p], acc_sh.at[dsts[p]], ss[p], add=True)
                fire_linear(w + 2, p)
            return 0

        lax.fori_loop(0, NWIN // 2, body2, 0)
        wait_linear(1)
        wait_gather(0)
        wait_scatter(0)
        wait_scatter(1)
        plsc.subcore_barrier()
        pltpu.sync_copy(
            acc_sh.at[pl.ds(sid * NROW, NROW)],
            out_hbm.at[cid, pl.ds(sid * NROW, NROW)],
        )

    return k(ytab_flat, cnt, src, dst, et)


def _extend_params(comp, basis, root):
    compe = jnp.zeros((54, 16), _F32).at[:R, :10].set(comp).at[R, 10].set(1.0)
    compe2 = (jnp.zeros((NPAIR, 32), _F32)
              .at[:, :16].set(compe[0::2])
              .at[:, 16:].set(compe[1::2]))
    basise2 = (jnp.zeros((32, H, H2), _F32)
               .at[:10, :, :H].set(basis)
               .at[10, :, :H].set(root)
               .at[16:26, :, H:].set(basis)
               .at[26, :, H:].set(root))
    return compe2, basise2.reshape(32, H * H2)


def _layer(x, compe, basise_flat, bias, cnt, src, dst, et, relu):
    w_all = _wmix(compe, basise_flat).reshape(NPAIR, H, H2)
    ytab = _tables(x, w_all)
    acc = _sc_edges(ytab.reshape(NPAIR * NN_PAD, H2), cnt, src, dst, et)
    return _combine(acc, ytab, bias, relu)


def kernel(x_drug, x_protein, edge_index, edge_type, Wd, bd, Wp, bp,
           comp1, basis1, root1, bias1, comp2, basis2, root2, bias2):
    h_d = _mm_bias(x_drug, Wd, bd, 400)
    h_p = _mm_bias(x_protein, Wp, bp, 400)
    x0 = jnp.concatenate(
        [h_d, h_p, jnp.zeros((NN_PAD - N_NODES, H), _F32)], axis=0
    )

    src = edge_index[0].astype(jnp.int32)
    dst = edge_index[1].astype(jnp.int32)
    et = edge_type.astype(jnp.int32)

    cnt2 = _sc_counts(dst, et)
    cnt = cnt2[0] + cnt2[1]

    c1e, b1e = _extend_params(comp1, basis1, root1)
    c2e, b2e = _extend_params(comp2, basis2, root2)

    x1 = _layer(x0, c1e, b1e, bias1, cnt, src, dst, et, relu=True)
    x2 = _layer(x1, c2e, b2e, bias2, cnt, src, dst, et, relu=False)

    return (x2[:N_DRUGS], x2[N_DRUGS:N_NODES])


# speedup vs baseline: 1.7628x; 1.7628x over previous
import functools

import jax
import jax.numpy as jnp
from jax import lax
from jax.experimental import pallas as pl
from jax.experimental.pallas import tpu as pltpu
from jax.experimental.pallas import tpu_sc as plsc

N_DRUGS = 2000
N_NODES = 10000
H = 64
R = 53
E = 640000
H2 = 128

NN_PAD = 10240
NKEY_PAD = 530432
ZCH = NKEY_PAD // 16
NROW = 640
NC, NS = 2, 16
NW = NC * NS
EW = E // NW
WIN = 80
NWIN = EW // WIN

_F32 = jnp.float32


def _sc_mesh():
    return plsc.VectorSubcoreMesh(
        core_axis_name="c", subcore_axis_name="s", num_cores=NC, num_subcores=NS
    )


def _mm_bias_body(x_ref, w_ref, b_ref, o_ref):
    o_ref[...] = (
        jnp.dot(x_ref[...], w_ref[...], preferred_element_type=_F32)
        + b_ref[...]
    )


def _mm_bias(x, w, b, bm):
    m, k = x.shape
    n = w.shape[1]
    return pl.pallas_call(
        _mm_bias_body,
        grid=(m // bm,),
        in_specs=[
            pl.BlockSpec((bm, k), lambda i: (i, 0)),
            pl.BlockSpec((k, n), lambda i: (0, 0)),
            pl.BlockSpec((1, n), lambda i: (0, 0)),
        ],
        out_specs=pl.BlockSpec((bm, n), lambda i: (i, 0)),
        out_shape=jax.ShapeDtypeStruct((m, n), _F32),
    )(x, w, b.reshape(1, n))


def _wmix_body(c_ref, b_ref, o_ref):
    o_ref[...] = jnp.dot(c_ref[...], b_ref[...], preferred_element_type=_F32)


def _wmix(compe, basise_flat):
    return pl.pallas_call(
        _wmix_body,
        out_shape=jax.ShapeDtypeStruct((54, H * H2), _F32),
    )(compe, basise_flat)


def _tables_body(w_ref, x_ref, o_ref):
    o_ref[0] = jnp.dot(x_ref[...], w_ref[0], preferred_element_type=_F32)


def _tables(x, w_all):
    return pl.pallas_call(
        _tables_body,
        grid=(54,),
        in_specs=[
            pl.BlockSpec((1, H, H2), lambda r: (r, 0, 0)),
            pl.BlockSpec((NN_PAD, H), lambda r: (0, 0)),
        ],
        out_specs=pl.BlockSpec((1, NN_PAD, H2), lambda r: (r, 0, 0)),
        out_shape=jax.ShapeDtypeStruct((54, NN_PAD, H2), _F32),
    )(w_all, x)


def _combine_body(acc_ref, yt_ref, b_ref, o_ref, *, relu):
    v = acc_ref[0] + acc_ref[1] + yt_ref[0, :, :H] + b_ref[...]
    o_ref[...] = jnp.maximum(v, 0.0) if relu else v


def _combine(acc, ytab, bias, relu):
    return pl.pallas_call(
        functools.partial(_combine_body, relu=relu),
        grid=(1,),
        in_specs=[
            pl.BlockSpec((2, NN_PAD, H), lambda i: (0, 0, 0)),
            pl.BlockSpec((1, NN_PAD, H2), lambda i: (53, 0, 0)),
            pl.BlockSpec((1, H), lambda i: (0, 0)),
        ],
        out_specs=pl.BlockSpec((NN_PAD, H), lambda i: (0, 0)),
        out_shape=jax.ShapeDtypeStruct((NN_PAD, H), _F32),
    )(acc, ytab, bias.reshape(1, H))


def _sc_counts(dst, et):
    @functools.partial(
        pl.kernel,
        out_type=jax.ShapeDtypeStruct((NC, NKEY_PAD), _F32),
        mesh=_sc_mesh(),
        compiler_params=pltpu.CompilerParams(use_tc_tiling_on_sc=False),
        scratch_types=[
            pltpu.VMEM_SHARED((NKEY_PAD,), _F32),
            pltpu.VMEM((ZCH,), _F32),
        ]
        + [pltpu.VMEM((WIN,), jnp.int32) for _ in range(6)]
        + [pltpu.VMEM((WIN,), _F32)]
        + [pltpu.SemaphoreType.DMA for _ in range(4)],
    )
    def k(dst_hbm, et_hbm, out_hbm, cnt_sh, zb,
          dstb0, dstb1, etb0, etb1, keyb0, keyb1, onesb,
          sl0, sl1, ss0, ss1):
        cid = lax.axis_index("c")
        sid = lax.axis_index("s")
        wid = sid * NC + cid
        base = wid * EW

        dstb = (dstb0, dstb1)
        etb = (etb0, etb1)
        keyb = (keyb0, keyb1)
        sl = (sl0, sl1)
        ss = (ss0, ss1)

        z16 = jnp.zeros((16,), _F32)
        o16 = jnp.ones((16,), _F32)

        def zbody(i, _):
            zb[pl.ds(i * 16, 16)] = z16
            return 0

        lax.fori_loop(0, ZCH // 16, zbody, 0)
        pltpu.sync_copy(zb, cnt_sh.at[pl.ds(sid * ZCH, ZCH)])
        for t in range(WIN // 16):
            onesb[pl.ds(16 * t, 16)] = o16
        plsc.subcore_barrier()

        def fire_linear(w, p):
            o = base + jnp.minimum(w, NWIN - 1) * WIN
            pltpu.async_copy(dst_hbm.at[pl.ds(o, WIN)], dstb[p], sl[p])
            pltpu.async_copy(et_hbm.at[pl.ds(o, WIN)], etb[p], sl[p])

        def wait_linear(p):
            pltpu.make_async_copy(dst_hbm.at[pl.ds(base, WIN)], dstb[p], sl[p]).wait()
            pltpu.make_async_copy(et_hbm.at[pl.ds(base, WIN)], etb[p], sl[p]).wait()

        def wait_scatter(p):
            pltpu.make_async_copy(onesb, cnt_sh.at[keyb[p]], ss[p]).wait()

        fire_linear(0, 0)
        fire_linear(1, 1)

        def body2(w2, _):
            for p in (0, 1):
                w = w2 * 2 + p
                wait_linear(p)
                @pl.when(w >= 2)
                def _():
                    wait_scatter(p)
                for t in range(WIN // 16):
                    sli = pl.ds(16 * t, 16)
                    keyb[p][sli] = dstb[p][sli] * R + etb[p][sli]
                pltpu.async_copy(onesb, cnt_sh.at[keyb[p]], ss[p], add=True)
                fire_linear(w + 2, p)
            return 0

        lax.fori_loop(0, NWIN // 2, body2, 0)
        wait_linear(0)
        wait_linear(1)
        wait_scatter(0)
        wait_scatter(1)
        plsc.subcore_barrier()
        pltpu.sync_copy(
            cnt_sh.at[pl.ds(sid * ZCH, ZCH)],
            out_hbm.at[cid, pl.ds(sid * ZCH, ZCH)],
        )

    return k(dst, et)


def _sc_edges(ytab_flat, cnt, src, dst, et):
    @functools.partial(
        pl.kernel,
        out_type=jax.ShapeDtypeStruct((NC, NN_PAD, H), _F32),
        mesh=_sc_mesh(),
        compiler_params=pltpu.CompilerParams(use_tc_tiling_on_sc=False),
        scratch_types=[
            pltpu.VMEM_SHARED((NN_PAD, H), _F32),
            pltpu.VMEM((NROW, H), _F32),
        ]
        + [pltpu.VMEM((WIN,), jnp.int32) for _ in range(12)]
        + [pltpu.VMEM((WIN,), _F32) for _ in range(3)]
        + [pltpu.VMEM((WIN, H2), _F32) for _ in range(2)]
        + [pltpu.VMEM((WIN, H), _F32) for _ in range(2)]
        + [pltpu.SemaphoreType.DMA for _ in range(6)],
    )
    def k(yt_hbm, cnt_hbm, src_hbm, dst_hbm, et_hbm, out_hbm,
          acc_sh, zb,
          srcb0, srcb1, dstb0, dstb1, etb0, etb1,
          rowb0, rowb1, keyb0, keyb1, dsts0, dsts1,
          cb0, cb1, invb, yb0, yb1, ysc0, ysc1,
          sl0, sl1, sg0, sg1, ss0, ss1):
        cid = lax.axis_index("c")
        sid = lax.axis_index("s")
        wid = sid * NC + cid
        base = wid * EW

        srcb = (srcb0, srcb1)
        dstb = (dstb0, dstb1)
        etb = (etb0, etb1)
        rowb = (rowb0, rowb1)
        keyb = (keyb0, keyb1)
        dsts = (dsts0, dsts1)
        cb = (cb0, cb1)
        yb = (yb0, yb1)
        ysc = (ysc0, ysc1)
        sl = (sl0, sl1)
        sg = (sg0, sg1)
        ss = (ss0, ss1)

        z16 = jnp.zeros((16,), _F32)
        zch = H // 16

        def zbody(i, _):
            zb[pl.ds(i // zch, 1), pl.ds((i % zch) * 16, 16)] = z16.reshape(1, 16)
            return 0

        lax.fori_loop(0, NROW * zch, zbody, 0)
        pltpu.sync_copy(zb, acc_sh.at[pl.ds(sid * NROW, NROW)])
        plsc.subcore_barrier()

        def off_(w):
            return base + jnp.minimum(w, NWIN - 1) * WIN

        def fire_linear(w, p):
            o = off_(w)
            pltpu.async_copy(src_hbm.at[pl.ds(o, WIN)], srcb[p], sl[p])
            pltpu.async_copy(dst_hbm.at[pl.ds(o, WIN)], dstb[p], sl[p])
            pltpu.async_copy(et_hbm.at[pl.ds(o, WIN)], etb[p], sl[p])

        def wait_linear(p):
            pltpu.make_async_copy(src_hbm.at[pl.ds(base, WIN)], srcb[p], sl[p]).wait()
            pltpu.make_async_copy(dst_hbm.at[pl.ds(base, WIN)], dstb[p], sl[p]).wait()
            pltpu.make_async_copy(et_hbm.at[pl.ds(base, WIN)], etb[p], sl[p]).wait()

        def keys_and_fire_gather(p):
            for t in range(WIN // 16):
                sli = pl.ds(16 * t, 16)
                rowb[p][sli] = etb[p][sli] * NN_PAD + srcb[p][sli]
                keyb[p][sli] = dstb[p][sli] * R + etb[p][sli]
            pltpu.async_copy(cnt_hbm.at[keyb[p]], cb[p], sg[p])
            pltpu.async_copy(yt_hbm.at[rowb[p]], yb[p], sg[p])

        def wait_gather(p):
            pltpu.make_async_copy(cnt_hbm.at[keyb[p]], cb[p], sg[p]).wait()
            pltpu.make_async_copy(yt_hbm.at[rowb[p]], yb[p], sg[p]).wait()

        def wait_scatter(p):
            pltpu.make_async_copy(ysc[p], acc_sh.at[dsts[p]], ss[p]).wait()

        fire_linear(0, 0)
        wait_linear(0)
        keys_and_fire_gather(0)
        fire_linear(1, 1)

        def body2(w2, _):
            for p in (0, 1):
                w = w2 * 2 + p
                q = 1 - p
                wait_linear(q)
                keys_and_fire_gather(q)
                @pl.when(w >= 2)
                def _():
                    wait_scatter(p)
                wait_gather(p)
                for t in range(WIN // 16):
                    sli = pl.ds(16 * t, 16)
                    invb[sli] = 1.0 / jnp.maximum(cb[p][sli], 1.0)
                    dsts[p][sli] = dstb[p][sli]

                def gbody(t, _):
                    iv16 = invb[pl.ds(16 * t, 16)]
                    for l in range(16):
                        sc = iv16[l]
                        for j in range(H // 16):
                            slx = (pl.ds(16 * t + l, 1), pl.ds(16 * j, 16))
                            ysc[p][slx] = yb[p][slx] * sc
                    return 0

                lax.fori_loop(0, WIN // 16, gbody, 0, unroll=WIN // 16)
                pltpu.async_copy(ysc[p], acc_sh.at[dsts[p]], ss[p], add=True)
                fire_linear(w + 2, p)
            return 0

        lax.fori_loop(0, NWIN // 2, body2, 0)
        wait_linear(1)
        wait_gather(0)
        wait_scatter(0)
        wait_scatter(1)
        plsc.subcore_barrier()
        pltpu.sync_copy(
            acc_sh.at[pl.ds(sid * NROW, NROW)],
            out_hbm.at[cid, pl.ds(sid * NROW, NROW)],
        )

    return k(ytab_flat, cnt, src, dst, et)


def _extend_params(comp, basis, root):
    compe = jnp.zeros((54, 16), _F32).at[:R, :10].set(comp).at[R, 10].set(1.0)
    basise = (jnp.zeros((16, H, H2), _F32)
              .at[:10, :, :H].set(basis)
              .at[10, :, :H].set(root))
    return compe, basise.reshape(16, H * H2)


def _layer(x, compe, basise_flat, bias, cnt, src, dst, et, relu):
    w_all = _wmix(compe, basise_flat).reshape(54, H, H2)
    ytab = _tables(x, w_all)
    acc = _sc_edges(ytab.reshape(54 * NN_PAD, H2), cnt, src, dst, et)
    return _combine(acc, ytab, bias, relu)


def kernel(x_drug, x_protein, edge_index, edge_type, Wd, bd, Wp, bp,
           comp1, basis1, root1, bias1, comp2, basis2, root2, bias2):
    h_d = _mm_bias(x_drug, Wd, bd, 400)
    h_p = _mm_bias(x_protein, Wp, bp, 400)
    x0 = jnp.concatenate(
        [h_d, h_p, jnp.zeros((NN_PAD - N_NODES, H), _F32)], axis=0
    )

    src = edge_index[0].astype(jnp.int32)
    dst = edge_index[1].astype(jnp.int32)
    et = edge_type.astype(jnp.int32)

    cnt2 = _sc_counts(dst, et)
    cnt = cnt2[0] + cnt2[1]

    c1e, b1e = _extend_params(comp1, basis1, root1)
    c2e, b2e = _extend_params(comp2, basis2, root2)

    x1 = _layer(x0, c1e, b1e, bias1, cnt, src, dst, et, relu=True)
    x2 = _layer(x1, c2e, b2e, bias2, cnt, src, dst, et, relu=False)

    return (x2[:N_DRUGS], x2[N_DRUGS:N_NODES])
